# baseline (speedup 1.0000x reference)
import functools

import jax
import jax.numpy as jnp
from jax import lax
from jax.experimental import pallas as pl
from jax.experimental.pallas import tpu as pltpu
from jax.experimental.pallas import tpu_sc as plsc

_N = 10000
_E = 320000
_G = 64
_NC = 2
_NS = 16
_NW = _NC * _NS
_CH = 128
_EP = 327680
_EPW = _EP // _NW
_NIT = _EPW // _CH
_NP = 10240
_RPT = _NP // _NS
_ZR = 80
_NPS = 10112
_RPTS = _NPS // _NS
_ZRS = 79

_BR = 2048
_NB = 5

_mesh = plsc.VectorSubcoreMesh(core_axis_name="c", subcore_axis_name="s")
_sc_params = pltpu.CompilerParams(use_tc_tiling_on_sc=False)


def _sc_degree(e):
    out = jax.ShapeDtypeStruct((_NC, _NP), jnp.float32)
    scratch = [
        pltpu.VMEM((_NIT, _CH), jnp.int32),
        pltpu.VMEM((_CH,), jnp.float32),
        pltpu.VMEM((_RPT,), jnp.float32),
        pltpu.VMEM_SHARED((_NP,), jnp.float32),
    ]

    @functools.partial(pl.kernel, out_type=out, mesh=_mesh,
                       scratch_types=scratch, compiler_params=_sc_params)
    def k(er, o, dstv, onev, zv, a):
        c = lax.axis_index("c")
        s = lax.axis_index("s")
        wid = c * _NS + s

        def fill_one(i, _):
            onev[pl.ds(i * 16, 16)] = jnp.ones((16,), jnp.float32)
            return 0
        lax.fori_loop(0, _CH // 16, fill_one, 0)

        def fill_zero(i, _):
            zv[pl.ds(i * 16, 16)] = jnp.zeros((16,), jnp.float32)
            return 0
        lax.fori_loop(0, _RPT // 16, fill_zero, 0)

        pltpu.sync_copy(zv, a.at[pl.ds(s * _RPT, _RPT)])
        pltpu.sync_copy(er.at[1, wid], dstv)
        plsc.subcore_barrier()

        def it(i, _):
            pltpu.sync_copy(onev, a.at[dstv.at[i]], add=True)
            return 0
        lax.fori_loop(0, _NIT, it, 0)
        plsc.subcore_barrier()

        pltpu.sync_copy(a.at[pl.ds(s * _RPT, _RPT)],
                        o.at[c, pl.ds(s * _RPT, _RPT)])

    return k(e)


def _sc_scatter(F, y, e):
    out = jax.ShapeDtypeStruct((_NC, _NPS, F), jnp.float32)
    _NBUF = 3
    scratch = (
        [pltpu.VMEM((_NIT, _CH), jnp.int32),
         pltpu.VMEM((_NIT, _CH), jnp.int32)]
        + [pltpu.VMEM((_CH, F), jnp.float32)] * _NBUF
        + [pltpu.VMEM((_ZR, F), jnp.float32)]
        + [pltpu.SemaphoreType.DMA] * (2 * _NBUF)
        + [pltpu.VMEM_SHARED((_NPS, F), jnp.float32),
           pltpu.VMEM_SHARED((_N, F), jnp.float32)]
    )

    @functools.partial(pl.kernel, out_type=out, mesh=_mesh,
                       scratch_types=scratch, compiler_params=_sc_params)
    def k(yr, er, o, srcv, dstv, r0, r1, r2, zrow,
          g0, g1, g2, s0, s1, s2, acc, ysh):
        rows = (r0, r1, r2)
        gs = (g0, g1, g2)
        ss = (s0, s1, s2)
        c = lax.axis_index("c")
        s = lax.axis_index("s")
        wid = c * _NS + s

        def zfill(r, _):
            for j in range(F // 16):
                zrow[r, pl.ds(j * 16, 16)] = jnp.zeros((16,), jnp.float32)
            return 0
        lax.fori_loop(0, _ZR, zfill, 0)

        def zero_own_slice():
            def cz(j, _):
                pltpu.sync_copy(zrow.at[pl.ds(0, _ZRS)],
                                acc.at[pl.ds(s * _RPTS + j * _ZRS, _ZRS)])
                return 0
            lax.fori_loop(0, _RPTS // _ZRS, cz, 0)

        zero_own_slice()
        _SR = _N // _NS
        pltpu.sync_copy(er.at[0, wid], srcv)
        pltpu.sync_copy(er.at[1, wid], dstv)
        pltpu.sync_copy(yr.at[pl.ds(s * _SR, _SR)],
                        ysh.at[pl.ds(s * _SR, _SR)])
        plsc.subcore_barrier()

        for kk in range(_NBUF):
            pltpu.async_copy(ysh.at[srcv.at[kk]], rows[kk], gs[kk])

        def it(q, _):
            i0 = _NBUF * q
            for kk in range(_NBUF):
                pltpu.make_async_copy(
                    ysh.at[srcv.at[i0]], rows[kk], gs[kk]).wait()
                pltpu.async_copy(rows[kk], acc.at[dstv.at[i0 + kk]],
                                 ss[kk], add=True)
            for kk in range(_NBUF):
                pltpu.make_async_copy(
                    rows[kk], acc.at[dstv.at[i0]], ss[kk]).wait()

                @pl.when(q < _NIT // _NBUF - 1)
                def _next(kk=kk):
                    pltpu.async_copy(ysh.at[srcv.at[i0 + _NBUF + kk]],
                                     rows[kk], gs[kk])
            return 0
        lax.fori_loop(0, _NIT // _NBUF, it, 0)
        plsc.subcore_barrier()
        pltpu.sync_copy(acc.at[pl.ds(s * _RPTS, _RPTS)],
                        o.at[c, pl.ds(s * _RPTS, _RPTS)])

    return k(y, e)


def _tc_y(x, degt, W1):
    def body(x_ref, dg_ref, w_ref, y_ref):
        dinv = lax.rsqrt(dg_ref[...].sum(axis=1, keepdims=True) + 1.0)
        xw = jnp.dot(x_ref[...], w_ref[...],
                     preferred_element_type=jnp.float32)
        y_ref[...] = xw * dinv

    return pl.pallas_call(
        body, grid=(_NB,),
        in_specs=[
            pl.BlockSpec((_BR, 128), lambda i: (i, 0)),
            pl.BlockSpec((_BR, 2), lambda i: (i, 0)),
            pl.BlockSpec((128, 64), lambda i: (0, 0)),
        ],
        out_specs=pl.BlockSpec((_BR, 64), lambda i: (i, 0)),
        out_shape=jax.ShapeDtypeStruct((_N, 64), jnp.float32),
    )(x, degt, W1)


def _tc_mid(p, y, degt, W2, b1):
    def body(p_ref, y_ref, dg_ref, w_ref, b_ref, o_ref):
        dinv = lax.rsqrt(dg_ref[...].sum(axis=1, keepdims=True) + 1.0)
        pre = p_ref[0] + p_ref[1] + y_ref[...]
        h = jnp.maximum(pre * dinv + b_ref[...], 0.0)
        o_ref[...] = jnp.dot(h, w_ref[...],
                             preferred_element_type=jnp.float32) * dinv

    return pl.pallas_call(
        body, grid=(_NB,),
        in_specs=[
            pl.BlockSpec((2, _BR, 64), lambda i: (0, i, 0)),
            pl.BlockSpec((_BR, 64), lambda i: (i, 0)),
            pl.BlockSpec((_BR, 2), lambda i: (i, 0)),
            pl.BlockSpec((64, 32), lambda i: (0, 0)),
            pl.BlockSpec((1, 64), lambda i: (0, 0)),
        ],
        out_specs=pl.BlockSpec((_BR, 32), lambda i: (i, 0)),
        out_shape=jax.ShapeDtypeStruct((_N, 32), jnp.float32),
    )(p, y, degt, W2, b1)


def _tc_head(p2, y2, degt, batch3, b2, fW1, fb1, fW2, fb2):
    def body(p_ref, y_ref, dg_ref, bt_ref, b2_ref,
             w1_ref, c1_ref, w2_ref, c2_ref, o_ref, acc):
        i = pl.program_id(0)

        @pl.when(i == 0)
        def _init():
            acc[...] = jnp.zeros((_G, 64), jnp.float32)

        dinv = lax.rsqrt(dg_ref[...].sum(axis=1, keepdims=True) + 1.0)
        o2 = (p_ref[0] + p_ref[1] + y_ref[...]) * dinv + b2_ref[...]
        rows = lax.broadcasted_iota(jnp.int32, (_BR, 1), 0) + i * _BR
        valid = (rows < _N).astype(jnp.float32)
        o2 = jnp.where(rows < _N, o2, 0.0)
        b = bt_ref[0, 0, :]
        P = (b[:, None] == lax.broadcasted_iota(jnp.int32, (_BR, _G), 1)
             ).astype(jnp.float32)
        ext = jnp.concatenate(
            [o2, valid, jnp.zeros((_BR, 31), jnp.float32)], axis=1)
        acc[...] += lax.dot_general(P, ext, (((0,), (0,)), ((), ())),
                                    preferred_element_type=jnp.float32)

        @pl.when(i == _NB - 1)
        def _fin():
            a = acc[...]
            pooled = a[:, :32] / jnp.maximum(a[:, 32:33], 1.0)
            z = jnp.maximum(
                jnp.dot(pooled, w1_ref[...],
                        preferred_element_type=jnp.float32) + c1_ref[...], 0.0)
            o_ref[...] = jnp.dot(z, w2_ref[...],
                                 preferred_element_type=jnp.float32) + c2_ref[...]

    return pl.pallas_call(
        body, grid=(_NB,),
        in_specs=[
            pl.BlockSpec((2, _BR, 32), lambda i: (0, i, 0)),
            pl.BlockSpec((_BR, 32), lambda i: (i, 0)),
            pl.BlockSpec((_BR, 2), lambda i: (i, 0)),
            pl.BlockSpec((1, 1, _BR), lambda i: (i, 0, 0)),
            pl.BlockSpec((1, 32), lambda i: (0, 0)),
            pl.BlockSpec((32, 32), lambda i: (0, 0)),
            pl.BlockSpec((1, 32), lambda i: (0, 0)),
            pl.BlockSpec((32, 32), lambda i: (0, 0)),
            pl.BlockSpec((1, 32), lambda i: (0, 0)),
        ],
        out_specs=pl.BlockSpec((_G, 32), lambda i: (0, 0)),
        out_shape=jax.ShapeDtypeStruct((_G, 32), jnp.float32),
        scratch_shapes=[pltpu.VMEM((_G, 64), jnp.float32)],
    )(p2, y2, degt, batch3, b2, fW1, fb1, fW2, fb2)


def kernel(x1, edge_index1, batch1, x2, edge_index2, batch2,
           x3, edge_index3, batch3, W1, b1, W2, b2, fW1, fb1, fW2, fb2):
    npad = _EP - _E
    src_pad = (jnp.arange(npad, dtype=jnp.int32) % _N)
    dst_pad = _N + (jnp.arange(npad, dtype=jnp.int32) % (_NPS - _N))
    epad = jnp.stack([src_pad, dst_pad])

    e1 = jnp.concatenate([edge_index1, epad], 1).reshape(2, _NW, _NIT, _CH)
    e2 = jnp.concatenate([edge_index2, epad], 1).reshape(2, _NW, _NIT, _CH)
    e3 = jnp.concatenate([edge_index3, epad], 1).reshape(2, _NW, _NIT, _CH)

    d1, d2, d3 = _sc_degree(e1), _sc_degree(e2), _sc_degree(e3)
    degt1, degt2, degt3 = d1.T, d2.T, d3.T

    y1 = _tc_y(x1, degt1, W1)
    y2 = _tc_y(x2, degt2, W1)
    y3 = _tc_y(x3, degt3, W1)

    p1 = _sc_scatter(64, y1, e1)
    p2 = _sc_scatter(64, y2, e2)
    p3 = _sc_scatter(64, y3, e3)

    b1r = b1.reshape(1, 64)
    z1 = _tc_mid(p1, y1, degt1, W2, b1r)
    z2 = _tc_mid(p2, y2, degt2, W2, b1r)
    z3 = _tc_mid(p3, y3, degt3, W2, b1r)

    q1 = _sc_scatter(32, z1, e1)
    q2 = _sc_scatter(32, z2, e2)
    q3 = _sc_scatter(32, z3, e3)

    pad = _NB * _BR - _N
    bt1 = jnp.pad(batch1, (0, pad), constant_values=_G).reshape(_NB, 1, _BR)
    bt2 = jnp.pad(batch2, (0, pad), constant_values=_G).reshape(_NB, 1, _BR)
    bt3 = jnp.pad(batch3, (0, pad), constant_values=_G).reshape(_NB, 1, _BR)
    b2r = b2.reshape(1, 32)
    fb1r = fb1.reshape(1, 32)
    fb2r = fb2.reshape(1, 32)

    o1 = _tc_head(q1, z1, degt1, bt1, b2r, fW1, fb1r, fW2, fb2r)
    o2 = _tc_head(q2, z2, degt2, bt2, b2r, fW1, fb1r, fW2, fb2r)
    o3 = _tc_head(q3, z3, degt3, bt3, b2r, fW1, fb1r, fW2, fb2r)
    return (o1, o2, o3)
